# baseline (speedup 1.0000x reference)
import functools

import jax
import jax.numpy as jnp
from jax import lax
from jax.experimental import pallas as pl
from jax.experimental.pallas import tpu as pltpu
from jax.experimental.pallas import tpu_sc as plsc

_NC = 2
_NS = 16
_NW = _NC * _NS
_BB = 128
_DP = 128


@functools.cache
def _make_lookup(L: int, B: int, D: int):
    assert B == _NW * _BB
    mesh = plsc.VectorSubcoreMesh(core_axis_name="c", subcore_axis_name="s")

    @functools.partial(
        pl.kernel,
        mesh=mesh,
        out_type=jax.ShapeDtypeStruct((L, D, B), jnp.float32),
        scratch_types=[
            pltpu.VMEM((2, _BB), jnp.int32),
            pltpu.VMEM((2, _BB, _DP), jnp.float32),
            pltpu.VMEM((2, D, _BB), jnp.float32),
            pltpu.SemaphoreType.DMA,
            pltpu.SemaphoreType.DMA,
            pltpu.SemaphoreType.DMA,
            pltpu.SemaphoreType.DMA,
        ],
        compiler_params=pltpu.CompilerParams(
            use_tc_tiling_on_sc=True, needs_layout_passes=False
        ),
    )
    def lookup(idx_hbm, table_hbm, out_hbm, idx_v, rows_v, pan_v, sg0, sg1, sp0, sp1):
        wid = lax.axis_index("s") * _NC + lax.axis_index("c")
        b0 = wid * _BB
        sg = (sg0, sg1)
        sp = (sp0, sp1)

        def load_and_fire(l, b):
            pltpu.sync_copy(idx_hbm.at[l, pl.ds(b0, _BB)], idx_v.at[b])
            pltpu.make_async_copy(
                table_hbm.at[idx_v.at[b]], rows_v.at[b], sg[b]
            ).start()

        def wait_gather(l, b):
            pltpu.make_async_copy(
                table_hbm.at[idx_v.at[b]], rows_v.at[b], sg[b]
            ).wait()

        def transpose_unit(b):
            rows = rows_v.at[b]
            pan = pan_v.at[b]
            lane = lax.iota(jnp.int32, 16)

            def d_body(d, carry):
                dcol = lane * 0 + d
                for bq in range(_BB // 16):
                    vec = plsc.load_gather(rows, [bq * 16 + lane, dcol])
                    pan[d, pl.ds(bq * 16, 16)] = vec
                return carry

            lax.fori_loop(0, D, d_body, 0, unroll=4)

        def fire_writeback(l, b):
            pltpu.make_async_copy(
                pan_v.at[b], out_hbm.at[l, :, pl.ds(b0, _BB)], sp[b]
            ).start()

        def wait_writeback(l, b):
            pltpu.make_async_copy(
                pan_v.at[b], out_hbm.at[l, :, pl.ds(b0, _BB)], sp[b]
            ).wait()

        load_and_fire(0, 0)

        def body2(i, carry):
            l0 = 2 * i
            l1 = l0 + 1
            load_and_fire(l1, 1)
            wait_gather(l0, 0)
            @pl.when(i > 0)
            def _():
                wait_writeback(l0 - 2, 0)
            transpose_unit(0)
            fire_writeback(l0, 0)
            @pl.when(l1 + 1 < L)
            def _():
                load_and_fire(l1 + 1, 0)
            wait_gather(l1, 1)
            @pl.when(i > 0)
            def _():
                wait_writeback(l1 - 2, 1)
            transpose_unit(1)
            fire_writeback(l1, 1)
            return carry

        lax.fori_loop(0, L // 2, body2, 0)
        wait_writeback(L - 2, 0)
        wait_writeback(L - 1, 1)

    return lookup


def kernel(token_ids, weight):
    bsz, seq = token_ids.shape
    n, d = weight.shape
    idx_t = token_ids.T.astype(jnp.int32)
    w_pad = jnp.pad(weight, ((0, 0), (0, _DP - d)))
    out_t = _make_lookup(seq, bsz, d)(idx_t, w_pad)
    return jnp.transpose(out_t, (2, 0, 1))


# speedup vs baseline: 1.8275x; 1.8275x over previous
import functools

import jax
import jax.numpy as jnp
from jax import lax
from jax.experimental import pallas as pl
from jax.experimental.pallas import tpu as pltpu
from jax.experimental.pallas import tpu_sc as plsc

_NC = 2
_NS = 16
_NW = _NC * _NS
_IW = 128
_K = 2
_SUP = _K * _IW


@functools.cache
def _make_lookup(B: int, DP: int):
    assert B % (_NW * _SUP) == 0
    b_per_w = B // _NW
    n_sup = b_per_w // _SUP
    assert n_sup % 2 == 0
    mesh = plsc.VectorSubcoreMesh(core_axis_name="c", subcore_axis_name="s")

    @functools.partial(
        pl.kernel,
        mesh=mesh,
        out_type=jax.ShapeDtypeStruct((B, DP), jnp.float32),
        scratch_types=[
            pltpu.VMEM((2, _K, _IW), jnp.int32),
            pltpu.VMEM((2, _SUP, DP), jnp.float32),
            pltpu.SemaphoreType.DMA,
            pltpu.SemaphoreType.DMA,
            pltpu.SemaphoreType.DMA,
            pltpu.SemaphoreType.DMA,
        ],
        compiler_params=pltpu.CompilerParams(use_tc_tiling_on_sc=True),
    )
    def lookup(idx_hbm, table_hbm, out_hbm, idx_v, rows_v, sg0, sg1, so0, so1):
        wid = lax.axis_index("s") * _NC + lax.axis_index("c")
        base = wid * b_per_w
        irow = wid * (b_per_w // _IW)
        sg = (sg0, sg1)
        so = (so0, so1)

        def load_and_fire(g, b):
            pltpu.sync_copy(idx_hbm.at[pl.ds(irow + g * _K, _K)], idx_v.at[b])
            for j in range(_K):
                pltpu.make_async_copy(
                    table_hbm.at[idx_v.at[b].at[j]],
                    rows_v.at[b].at[pl.ds(j * _IW, _IW)],
                    sg[b],
                ).start()

        def drain_and_writeback(g, b):
            pltpu.make_async_copy(
                out_hbm.at[pl.ds(base + g * _SUP, _SUP)], rows_v.at[b], sg[b]
            ).wait()
            pltpu.make_async_copy(
                rows_v.at[b], out_hbm.at[pl.ds(base + g * _SUP, _SUP)], so[b]
            ).start()

        def wait_writeback(g, b):
            pltpu.make_async_copy(
                rows_v.at[b], out_hbm.at[pl.ds(base + g * _SUP, _SUP)], so[b]
            ).wait()

        load_and_fire(0, 0)
        load_and_fire(1, 1)
        drain_and_writeback(0, 0)

        def body(i, carry):
            g0 = 2 * i
            g1 = g0 + 1
            wait_writeback(g0 - 2, 0)
            load_and_fire(g0, 0)
            drain_and_writeback(g0 - 1, 1)
            wait_writeback(g1 - 2, 1)
            load_and_fire(g1, 1)
            drain_and_writeback(g0, 0)
            return carry

        lax.fori_loop(1, n_sup // 2, body, 0)

        wait_writeback(n_sup - 2, 0)
        drain_and_writeback(n_sup - 1, 1)
        wait_writeback(n_sup - 1, 1)

    return lookup


def kernel(token_ids, weight):
    bsz, seq = token_ids.shape
    n, d = weight.shape
    idx2d = token_ids.reshape(-1, _IW).astype(jnp.int32)
    w_pad = jnp.pad(weight, ((0, 0), (0, _IW - d)))
    out = _make_lookup(bsz * seq, _IW)(idx2d, w_pad)
    return out[:, :d].reshape(bsz, seq, d)
